# Initial kernel scaffold; baseline (speedup 1.0000x reference)
import jax
import jax.numpy as jnp
from jax.experimental import pallas as pl


def kernel(src, rel, dst, ent_table, rel_table, W, b):
    raise NotImplementedError("write your pallas kernel here")


# speedup vs baseline: 1.5224x; 1.5224x over previous
import functools

import jax
import jax.numpy as jnp
from jax import lax
from jax.experimental import pallas as pl
from jax.experimental.pallas import tpu as pltpu
from jax.experimental.pallas import tpu_sc as plsc

NUM_ENT = 100000
NUM_REL = 116
DIM = 128
B = 16384

LANES = 16
CHUNK = 128


def _make_sc_comp():
    info = plsc.get_sparse_core_info()
    nc, ns = info.num_cores, info.num_subcores
    nw = nc * ns
    b_per_w = B // nw
    n_chunks = b_per_w // CHUNK
    mesh = plsc.VectorSubcoreMesh(core_axis_name="c", subcore_axis_name="s")

    @functools.partial(
        pl.kernel,
        mesh=mesh,
        out_type=jax.ShapeDtypeStruct((B, DIM), jnp.float32),
        scratch_types=[
            pltpu.VMEM((CHUNK,), jnp.int32),
            pltpu.VMEM((CHUNK,), jnp.int32),
            pltpu.VMEM((CHUNK,), jnp.int32),
            pltpu.VMEM((CHUNK, DIM), jnp.float32),
            pltpu.VMEM((CHUNK, DIM), jnp.float32),
            pltpu.VMEM((CHUNK, DIM), jnp.float32),
            pltpu.SemaphoreType.DMA,
        ],
    )
    def sc_comp(src_hbm, rel_hbm, dst_hbm, ent_hbm, relt_hbm, out_hbm,
                si_v, ri_v, di_v, h_v, r_v, t_v, sem):
        wid = lax.axis_index("s") * nc + lax.axis_index("c")
        base = wid * b_per_w
        for j in range(n_chunks):
            off = base + j * CHUNK
            pltpu.sync_copy(src_hbm.at[pl.ds(off, CHUNK)], si_v)
            pltpu.sync_copy(rel_hbm.at[pl.ds(off, CHUNK)], ri_v)
            pltpu.sync_copy(dst_hbm.at[pl.ds(off, CHUNK)], di_v)
            ch = pltpu.async_copy(ent_hbm.at[si_v], h_v, sem)
            cr = pltpu.async_copy(relt_hbm.at[ri_v], r_v, sem)
            ct = pltpu.async_copy(ent_hbm.at[di_v], t_v, sem)
            ch.wait()
            cr.wait()
            ct.wait()

            def row_body(i, carry):
                for c in range(DIM // LANES):
                    sl = pl.ds(c * LANES, LANES)
                    h_v[i, sl] = (h_v[i, sl] + r_v[i, sl]) - t_v[i, sl]
                return carry

            lax.fori_loop(0, CHUNK, row_body, 0)
            pltpu.sync_copy(h_v, out_hbm.at[pl.ds(off, CHUNK)])

    return sc_comp


_sc_comp = _make_sc_comp()


def _tc_body(comp_ref, w_ref, b_ref, out_ref):
    acc = jnp.dot(comp_ref[...], w_ref[...], preferred_element_type=jnp.float32)
    out_ref[...] = jnp.tanh(acc + b_ref[...])


def kernel(src, rel, dst, ent_table, rel_table, W, b):
    comp = _sc_comp(src, rel, dst, ent_table, rel_table)
    blk = 2048
    out = pl.pallas_call(
        _tc_body,
        grid=(B // blk,),
        in_specs=[
            pl.BlockSpec((blk, DIM), lambda i: (i, 0)),
            pl.BlockSpec((DIM, DIM), lambda i: (0, 0)),
            pl.BlockSpec((1, DIM), lambda i: (0, 0)),
        ],
        out_specs=pl.BlockSpec((blk, DIM), lambda i: (i, 0)),
        out_shape=jax.ShapeDtypeStruct((B, DIM), jnp.float32),
    )(comp, W, b.reshape(1, DIM))
    return out


# speedup vs baseline: 1.6631x; 1.0924x over previous
import functools

import jax
import jax.numpy as jnp
from jax import lax
from jax.experimental import pallas as pl
from jax.experimental.pallas import tpu as pltpu
from jax.experimental.pallas import tpu_sc as plsc

NUM_ENT = 100000
NUM_REL = 116
DIM = 128
B = 16384

LANES = 16
CHUNK = 128


def _make_sc_comp(rows):
    info = plsc.get_sparse_core_info()
    nc, ns = info.num_cores, info.num_subcores
    nw = nc * ns
    b_per_w = rows // nw
    n_chunks = b_per_w // CHUNK
    mesh = plsc.VectorSubcoreMesh(core_axis_name="c", subcore_axis_name="s")

    @functools.partial(
        pl.kernel,
        mesh=mesh,
        out_type=jax.ShapeDtypeStruct((rows, DIM), jnp.float32),
        scratch_types=[
            pltpu.VMEM((n_chunks, CHUNK), jnp.int32),
            pltpu.VMEM((n_chunks, CHUNK), jnp.int32),
            pltpu.VMEM((n_chunks, CHUNK), jnp.int32),
            pltpu.VMEM((2, CHUNK, DIM), jnp.float32),
            pltpu.VMEM((2, CHUNK, DIM), jnp.float32),
            pltpu.VMEM((2, CHUNK, DIM), jnp.float32),
            pltpu.SemaphoreType.DMA,
            pltpu.SemaphoreType.DMA,
            pltpu.SemaphoreType.DMA,
            pltpu.SemaphoreType.DMA,
        ],
    )
    def sc_comp(src_hbm, rel_hbm, dst_hbm, ent_hbm, relt_hbm, out_hbm,
                si_v, ri_v, di_v, h_v, r_v, t_v, g0, g1, o0, o1):
        wid = lax.axis_index("s") * nc + lax.axis_index("c")
        wc0 = wid * n_chunks
        base = wid * b_per_w
        gsem = (g0, g1)
        osem = (o0, o1)

        pltpu.sync_copy(src_hbm.at[pl.ds(wc0, n_chunks)], si_v)
        pltpu.sync_copy(rel_hbm.at[pl.ds(wc0, n_chunks)], ri_v)
        pltpu.sync_copy(dst_hbm.at[pl.ds(wc0, n_chunks)], di_v)

        def start_gather(j, s):
            sem = gsem[s]
            return (
                pltpu.async_copy(ent_hbm.at[si_v.at[j]], h_v.at[s], sem),
                pltpu.async_copy(relt_hbm.at[ri_v.at[j]], r_v.at[s], sem),
                pltpu.async_copy(ent_hbm.at[di_v.at[j]], t_v.at[s], sem),
            )

        gathers = [None, None]
        outs = [None, None]
        gathers[0] = start_gather(0, 0)
        for j in range(n_chunks):
            s = j & 1
            if j + 1 < n_chunks:
                if outs[1 - s] is not None:
                    outs[1 - s].wait()
                    outs[1 - s] = None
                gathers[1 - s] = start_gather(j + 1, 1 - s)
            for c in gathers[s]:
                c.wait()

            def row_body(i, carry, s=s):
                for c in range(DIM // LANES):
                    sl = pl.ds(c * LANES, LANES)
                    h_v[s, i, sl] = (h_v[s, i, sl] + r_v[s, i, sl]) - t_v[s, i, sl]
                return carry

            lax.fori_loop(0, CHUNK, row_body, 0)
            outs[s] = pltpu.async_copy(
                h_v.at[s], out_hbm.at[pl.ds(base + j * CHUNK, CHUNK)], osem[s])
        for s in range(2):
            if outs[s] is not None:
                outs[s].wait()

    return sc_comp


_sc_comp = _make_sc_comp(B)


def _tc_body(comp_ref, w_ref, b_ref, out_ref):
    acc = jnp.dot(comp_ref[...], w_ref[...], preferred_element_type=jnp.float32)
    out_ref[...] = jnp.tanh(acc + b_ref[...])


def kernel(src, rel, dst, ent_table, rel_table, W, b):
    nrc = B // CHUNK
    comp = _sc_comp(src.reshape(nrc, CHUNK), rel.reshape(nrc, CHUNK),
                    dst.reshape(nrc, CHUNK), ent_table, rel_table)
    blk = 2048
    out = pl.pallas_call(
        _tc_body,
        grid=(B // blk,),
        in_specs=[
            pl.BlockSpec((blk, DIM), lambda i: (i, 0)),
            pl.BlockSpec((DIM, DIM), lambda i: (0, 0)),
            pl.BlockSpec((1, DIM), lambda i: (0, 0)),
        ],
        out_specs=pl.BlockSpec((blk, DIM), lambda i: (i, 0)),
        out_shape=jax.ShapeDtypeStruct((B, DIM), jnp.float32),
    )(comp, W, b.reshape(1, DIM))
    return out
